# baseline (speedup 1.0000x reference)
import functools

import jax
import jax.numpy as jnp
from jax import lax
from jax.experimental import pallas as pl
from jax.experimental.pallas import tpu as pltpu
from jax.experimental.pallas import tpu_sc as plsc

N = 10000
E = 320000
H = 128
HW = H // 2
NC = 2
NS = 16
EB = 100
EPT_P = E // NS
NB_P = EPT_P // EB
EPT_D = E // (NC * NS)
NB_D = EPT_D // EB
NP = 10240
RPT = NP // NS
DW = 16

_mesh = plsc.VectorSubcoreMesh(
    core_axis_name="c", subcore_axis_name="s", num_cores=NC, num_subcores=NS
)


@functools.partial(
    pl.kernel,
    out_type=jax.ShapeDtypeStruct((NC, NP, DW), jnp.float32),
    mesh=_mesh,
    compiler_params=pltpu.CompilerParams(use_tc_tiling_on_sc=False),
    scratch_types=[
        pltpu.VMEM((NB_D, EB), jnp.int32),
        pltpu.VMEM((EB, DW), jnp.float32),
        pltpu.VMEM((RPT, DW), jnp.float32),
        pltpu.VMEM_SHARED((NP, DW), jnp.float32),
        pltpu.SemaphoreType.DMA,
    ],
)
def _deg_kernel(dst_hbm, out_hbm, didx, ones_v, zb, acc, sem):
    c = lax.axis_index("c")
    s = lax.axis_index("s")
    w = c * NS + s

    def _fill(i, _):
        ones_v[i, :] = jnp.full((DW,), 1.0, jnp.float32)
        return 0

    lax.fori_loop(0, EB, _fill, 0)

    def _zrow(i, _):
        zb[i, :] = jnp.zeros((DW,), jnp.float32)
        return 0

    lax.fori_loop(0, RPT, _zrow, 0)
    pltpu.sync_copy(zb, acc.at[pl.ds(s * RPT, RPT)])
    pltpu.sync_copy(dst_hbm.at[w], didx)
    plsc.subcore_barrier()

    DK = 25

    def _dgroup(g, _):
        base = g * DK
        for b in range(DK):
            pltpu.make_async_copy(ones_v, acc.at[didx.at[base + b]], sem).start(add=True)
        for b in range(DK):
            pltpu.make_async_copy(ones_v, acc.at[didx.at[0]], sem).wait()
        return 0

    lax.fori_loop(0, NB_D // DK, _dgroup, 0)
    plsc.subcore_barrier()
    pltpu.sync_copy(acc.at[pl.ds(s * RPT, RPT)], out_hbm.at[c, pl.ds(s * RPT, RPT), :])


NBUF = 5
NG = NB_P // NBUF


@functools.partial(
    pl.kernel,
    out_type=jax.ShapeDtypeStruct((NC, NP, HW), jnp.float32),
    mesh=_mesh,
    compiler_params=pltpu.CompilerParams(use_tc_tiling_on_sc=False),
    scratch_types=[
        pltpu.VMEM((NB_P, EB), jnp.int32),
        pltpu.VMEM((NB_P, EB), jnp.int32),
        pltpu.VMEM((NBUF, EB, HW), jnp.float32),
        pltpu.VMEM((RPT // 8, HW), jnp.float32),
        pltpu.VMEM_SHARED((NP, HW), jnp.float32),
        pltpu.SemaphoreType.DMA((NBUF,)),
        pltpu.SemaphoreType.DMA,
    ],
)
def _prop_kernel(y_hbm, ei_hbm, out_hbm, sidx, didx, rows, zb, acc, sem, psem):
    c = lax.axis_index("c")
    s = lax.axis_index("s")

    ps = pltpu.make_async_copy(ei_hbm.at[0, s], sidx, psem)
    ps.start()
    pd = pltpu.make_async_copy(ei_hbm.at[1, s], didx, psem)
    pd.start()

    def _zrow(i, _):
        for j in range(HW // 16):
            zb[i, pl.ds(j * 16, 16)] = jnp.zeros((16,), jnp.float32)
        return 0

    lax.fori_loop(0, RPT // 8, _zrow, 0)
    ps.wait()
    pd.wait()

    ytab = y_hbm.at[c]

    def g_start(b, j):
        pltpu.make_async_copy(ytab.at[sidx.at[j]], rows.at[b], sem.at[b]).start()

    def g_wait(b):
        pltpu.make_async_copy(ytab.at[sidx.at[0]], rows.at[b], sem.at[b]).wait()

    def s_start(b, j):
        pltpu.make_async_copy(rows.at[b], acc.at[didx.at[j]], sem.at[b]).start(add=True)

    def s_wait(b):
        pltpu.make_async_copy(rows.at[b], acc.at[didx.at[0]], sem.at[b]).wait()

    for b in range(NBUF):
        g_start(b, b)
    for r in range(8):
        pltpu.sync_copy(zb, acc.at[pl.ds(s * RPT + r * (RPT // 8), RPT // 8)])
    plsc.subcore_barrier()

    def _group(g, _):
        base = g * NBUF
        for b in range(NBUF):
            g_wait(b)
            s_start(b, base + b)
        for b in range(NBUF):
            s_wait(b)
            g_start(b, base + NBUF + b)
        return 0

    lax.fori_loop(0, NG - 1, _group, 0)
    last = (NG - 1) * NBUF
    for b in range(NBUF):
        g_wait(b)
        s_start(b, last + b)
    for b in range(NBUF):
        s_wait(b)
    plsc.subcore_barrier()
    pltpu.sync_copy(acc.at[pl.ds(s * RPT, RPT)], out_hbm.at[c, pl.ds(s * RPT, RPT), :])


_R = 1000
_C = 40


def _stage_pre_body(x_ref, w_ref, u_ref):
    u_ref[...] = jnp.dot(x_ref[...], w_ref[...], preferred_element_type=jnp.float32)


def _stage_scale_body(dp_ref, u_ref, y_ref, inv_ref):
    deg = dp_ref[0, :, 0:1] + dp_ref[1, :, 0:1] + 1.0
    inv = lax.rsqrt(deg)
    y = u_ref[...] * inv
    y_ref[0] = y[:, :HW]
    y_ref[1] = y[:, HW:]
    inv_ref[...] = inv


def _stage_mid_body(sp_ref, y_ref, inv_ref, b_ref, w_ref, y2_ref):
    s = jnp.concatenate([sp_ref[0], sp_ref[1]], axis=1)
    y = jnp.concatenate([y_ref[0], y_ref[1]], axis=1)
    inv = inv_ref[...]
    h = jnp.maximum(inv * (s + y) + b_ref[...], 0.0)
    y2 = jnp.dot(h, w_ref[...], preferred_element_type=jnp.float32) * inv
    y2_ref[0] = y2[:, :HW]
    y2_ref[1] = y2[:, HW:]


def _stage_out_body(sp_ref, y_ref, inv_ref, b_ref, w3_ref, b3_ref, o_ref):
    s = jnp.concatenate([sp_ref[0], sp_ref[1]], axis=1)
    y = jnp.concatenate([y_ref[0], y_ref[1]], axis=1)
    inv = inv_ref[...]
    h = jnp.maximum(inv * (s + y) + b_ref[...], 0.0)
    logits = jnp.dot(h, w3_ref[...], preferred_element_type=jnp.float32) + b3_ref[...]
    m = jnp.max(logits, axis=1, keepdims=True)
    e = jnp.exp(logits - m)
    o_ref[...] = e / jnp.sum(e, axis=1, keepdims=True)


_stage_pre = pl.pallas_call(
    _stage_pre_body,
    grid=(N // _R,),
    in_specs=[
        pl.BlockSpec((_R, H), lambda i: (i, 0)),
        pl.BlockSpec((H, H), lambda i: (0, 0)),
    ],
    out_specs=pl.BlockSpec((_R, H), lambda i: (i, 0)),
    out_shape=jax.ShapeDtypeStruct((N, H), jnp.float32),
)

_stage_scale = pl.pallas_call(
    _stage_scale_body,
    grid=(N // _R,),
    in_specs=[
        pl.BlockSpec((NC, _R, DW), lambda i: (0, i, 0)),
        pl.BlockSpec((_R, H), lambda i: (i, 0)),
    ],
    out_specs=[
        pl.BlockSpec((NC, _R, HW), lambda i: (0, i, 0)),
        pl.BlockSpec((_R, 1), lambda i: (i, 0)),
    ],
    out_shape=[
        jax.ShapeDtypeStruct((NC, N, HW), jnp.float32),
        jax.ShapeDtypeStruct((N, 1), jnp.float32),
    ],
)

_stage_mid = pl.pallas_call(
    _stage_mid_body,
    grid=(N // _R,),
    in_specs=[
        pl.BlockSpec((NC, _R, HW), lambda i: (0, i, 0)),
        pl.BlockSpec((NC, _R, HW), lambda i: (0, i, 0)),
        pl.BlockSpec((_R, 1), lambda i: (i, 0)),
        pl.BlockSpec((1, H), lambda i: (0, 0)),
        pl.BlockSpec((H, H), lambda i: (0, 0)),
    ],
    out_specs=pl.BlockSpec((NC, _R, HW), lambda i: (0, i, 0)),
    out_shape=jax.ShapeDtypeStruct((NC, N, HW), jnp.float32),
)

_stage_out = pl.pallas_call(
    _stage_out_body,
    grid=(N // _R,),
    in_specs=[
        pl.BlockSpec((NC, _R, HW), lambda i: (0, i, 0)),
        pl.BlockSpec((NC, _R, HW), lambda i: (0, i, 0)),
        pl.BlockSpec((_R, 1), lambda i: (i, 0)),
        pl.BlockSpec((1, H), lambda i: (0, 0)),
        pl.BlockSpec((H, _C), lambda i: (0, 0)),
        pl.BlockSpec((1, _C), lambda i: (0, 0)),
    ],
    out_specs=pl.BlockSpec((_R, _C), lambda i: (i, 0)),
    out_shape=jax.ShapeDtypeStruct((N, _C), jnp.float32),
)


def kernel(x, edge_index, W1, b1, W2, b2, W3, b3):
    ei_p = edge_index.reshape(2, NS, NB_P, EB)
    ei_d = edge_index[1].reshape(NC * NS, NB_D, EB)
    degp = _deg_kernel(ei_d)
    u1 = _stage_pre(x, W1)
    y1, inv = _stage_scale(degp, u1)
    s1 = _prop_kernel(y1, ei_p)
    y2 = _stage_mid(s1, y1, inv, b1.reshape(1, H), W2)
    s2 = _prop_kernel(y2, ei_p)
    return _stage_out(s2, y2, inv, b2.reshape(1, H), W3, b3.reshape(1, _C))


# speedup vs baseline: 1.0425x; 1.0425x over previous
import functools

import jax
import jax.numpy as jnp
from jax import lax
from jax.experimental import pallas as pl
from jax.experimental.pallas import tpu as pltpu
from jax.experimental.pallas import tpu_sc as plsc

N = 10000
E = 320000
H = 128
HW = H // 2
NC = 2
NS = 16
EB = 80
EPT_P = E // NS
NB_P = EPT_P // EB
EPT_D = E // (NC * NS)
NB_D = EPT_D // EB
NP = 10240
RPT = NP // NS
DW = 16

_mesh = plsc.VectorSubcoreMesh(
    core_axis_name="c", subcore_axis_name="s", num_cores=NC, num_subcores=NS
)


@functools.partial(
    pl.kernel,
    out_type=jax.ShapeDtypeStruct((NC, NP, DW), jnp.float32),
    mesh=_mesh,
    compiler_params=pltpu.CompilerParams(use_tc_tiling_on_sc=False),
    scratch_types=[
        pltpu.VMEM((NB_D, EB), jnp.int32),
        pltpu.VMEM((EB, DW), jnp.float32),
        pltpu.VMEM((RPT, DW), jnp.float32),
        pltpu.VMEM_SHARED((NP, DW), jnp.float32),
        pltpu.SemaphoreType.DMA,
    ],
)
def _deg_kernel(dst_hbm, out_hbm, didx, ones_v, zb, acc, sem):
    c = lax.axis_index("c")
    s = lax.axis_index("s")
    w = c * NS + s

    def _fill(i, _):
        ones_v[i, :] = jnp.full((DW,), 1.0, jnp.float32)
        return 0

    lax.fori_loop(0, EB, _fill, 0)

    def _zrow(i, _):
        zb[i, :] = jnp.zeros((DW,), jnp.float32)
        return 0

    lax.fori_loop(0, RPT, _zrow, 0)
    pltpu.sync_copy(zb, acc.at[pl.ds(s * RPT, RPT)])
    pltpu.sync_copy(dst_hbm.at[w], didx)
    plsc.subcore_barrier()

    DK = 25

    def _dgroup(g, _):
        base = g * DK
        for b in range(DK):
            pltpu.make_async_copy(ones_v, acc.at[didx.at[base + b]], sem).start(add=True)
        for b in range(DK):
            pltpu.make_async_copy(ones_v, acc.at[didx.at[0]], sem).wait()
        return 0

    lax.fori_loop(0, NB_D // DK, _dgroup, 0)
    plsc.subcore_barrier()
    pltpu.sync_copy(acc.at[pl.ds(s * RPT, RPT)], out_hbm.at[c, pl.ds(s * RPT, RPT), :])


NBUF = 5
NG = NB_P // NBUF


@functools.partial(
    pl.kernel,
    out_type=jax.ShapeDtypeStruct((NC, NP, HW), jnp.float32),
    mesh=_mesh,
    compiler_params=pltpu.CompilerParams(use_tc_tiling_on_sc=False),
    scratch_types=[
        pltpu.VMEM((NB_P, EB), jnp.int32),
        pltpu.VMEM((NB_P, EB), jnp.int32),
        pltpu.VMEM((NBUF, EB, HW), jnp.float32),
        pltpu.VMEM((RPT // 8, HW), jnp.float32),
        pltpu.VMEM_SHARED((NP, HW), jnp.float32),
        pltpu.SemaphoreType.DMA((NBUF,)),
        pltpu.SemaphoreType.DMA,
    ],
)
def _prop_kernel(y_hbm, ei_hbm, out_hbm, sidx, didx, rows, zb, acc, sem, psem):
    c = lax.axis_index("c")
    s = lax.axis_index("s")

    ps = pltpu.make_async_copy(ei_hbm.at[0, s], sidx, psem)
    ps.start()
    pd = pltpu.make_async_copy(ei_hbm.at[1, s], didx, psem)
    pd.start()

    def _zrow(i, _):
        for j in range(HW // 16):
            zb[i, pl.ds(j * 16, 16)] = jnp.zeros((16,), jnp.float32)
        return 0

    lax.fori_loop(0, RPT // 8, _zrow, 0)
    ps.wait()
    pd.wait()

    ytab = y_hbm.at[c]

    def g_start(b, j):
        pltpu.make_async_copy(ytab.at[sidx.at[j]], rows.at[b], sem.at[b]).start()

    def g_wait(b):
        pltpu.make_async_copy(ytab.at[sidx.at[0]], rows.at[b], sem.at[b]).wait()

    def s_start(b, j):
        pltpu.make_async_copy(rows.at[b], acc.at[didx.at[j]], sem.at[b]).start(add=True)

    def s_wait(b):
        pltpu.make_async_copy(rows.at[b], acc.at[didx.at[0]], sem.at[b]).wait()

    for b in range(NBUF):
        g_start(b, b)
    for r in range(8):
        pltpu.sync_copy(zb, acc.at[pl.ds(s * RPT + r * (RPT // 8), RPT // 8)])
    plsc.subcore_barrier()

    def _group(g, _):
        base = g * NBUF
        for b in range(NBUF):
            g_wait(b)
            s_start(b, base + b)
        for b in range(NBUF):
            s_wait(b)
            g_start(b, base + NBUF + b)
        return 0

    lax.fori_loop(0, NG - 1, _group, 0)
    last = (NG - 1) * NBUF
    for b in range(NBUF):
        g_wait(b)
        s_start(b, last + b)
    for b in range(NBUF):
        s_wait(b)
    plsc.subcore_barrier()
    pltpu.sync_copy(acc.at[pl.ds(s * RPT, RPT)], out_hbm.at[c, pl.ds(s * RPT, RPT), :])


_R = 2000
_C = 40


def _stage_pre_body(x_ref, w_ref, u_ref):
    u_ref[...] = jnp.dot(x_ref[...], w_ref[...], preferred_element_type=jnp.float32)


def _stage_scale_body(dp_ref, u_ref, y_ref, inv_ref):
    deg = dp_ref[0, :, 0:1] + dp_ref[1, :, 0:1] + 1.0
    inv = lax.rsqrt(deg)
    y = u_ref[...] * inv
    y_ref[0] = y[:, :HW]
    y_ref[1] = y[:, HW:]
    inv_ref[...] = inv


def _stage_mid_body(sp_ref, y_ref, inv_ref, b_ref, w_ref, y2_ref):
    s = jnp.concatenate([sp_ref[0], sp_ref[1]], axis=1)
    y = jnp.concatenate([y_ref[0], y_ref[1]], axis=1)
    inv = inv_ref[...]
    h = jnp.maximum(inv * (s + y) + b_ref[...], 0.0)
    y2 = jnp.dot(h, w_ref[...], preferred_element_type=jnp.float32) * inv
    y2_ref[0] = y2[:, :HW]
    y2_ref[1] = y2[:, HW:]


def _stage_out_body(sp_ref, y_ref, inv_ref, b_ref, w3_ref, b3_ref, o_ref):
    s = jnp.concatenate([sp_ref[0], sp_ref[1]], axis=1)
    y = jnp.concatenate([y_ref[0], y_ref[1]], axis=1)
    inv = inv_ref[...]
    h = jnp.maximum(inv * (s + y) + b_ref[...], 0.0)
    logits = jnp.dot(h, w3_ref[...], preferred_element_type=jnp.float32) + b3_ref[...]
    m = jnp.max(logits, axis=1, keepdims=True)
    e = jnp.exp(logits - m)
    o_ref[...] = e / jnp.sum(e, axis=1, keepdims=True)


_stage_pre = pl.pallas_call(
    _stage_pre_body,
    grid=(N // _R,),
    in_specs=[
        pl.BlockSpec((_R, H), lambda i: (i, 0)),
        pl.BlockSpec((H, H), lambda i: (0, 0)),
    ],
    out_specs=pl.BlockSpec((_R, H), lambda i: (i, 0)),
    out_shape=jax.ShapeDtypeStruct((N, H), jnp.float32),
)

_stage_scale = pl.pallas_call(
    _stage_scale_body,
    grid=(N // _R,),
    in_specs=[
        pl.BlockSpec((NC, _R, DW), lambda i: (0, i, 0)),
        pl.BlockSpec((_R, H), lambda i: (i, 0)),
    ],
    out_specs=[
        pl.BlockSpec((NC, _R, HW), lambda i: (0, i, 0)),
        pl.BlockSpec((_R, 1), lambda i: (i, 0)),
    ],
    out_shape=[
        jax.ShapeDtypeStruct((NC, N, HW), jnp.float32),
        jax.ShapeDtypeStruct((N, 1), jnp.float32),
    ],
)

_stage_mid = pl.pallas_call(
    _stage_mid_body,
    grid=(N // _R,),
    in_specs=[
        pl.BlockSpec((NC, _R, HW), lambda i: (0, i, 0)),
        pl.BlockSpec((NC, _R, HW), lambda i: (0, i, 0)),
        pl.BlockSpec((_R, 1), lambda i: (i, 0)),
        pl.BlockSpec((1, H), lambda i: (0, 0)),
        pl.BlockSpec((H, H), lambda i: (0, 0)),
    ],
    out_specs=pl.BlockSpec((NC, _R, HW), lambda i: (0, i, 0)),
    out_shape=jax.ShapeDtypeStruct((NC, N, HW), jnp.float32),
)

_stage_out = pl.pallas_call(
    _stage_out_body,
    grid=(N // _R,),
    in_specs=[
        pl.BlockSpec((NC, _R, HW), lambda i: (0, i, 0)),
        pl.BlockSpec((NC, _R, HW), lambda i: (0, i, 0)),
        pl.BlockSpec((_R, 1), lambda i: (i, 0)),
        pl.BlockSpec((1, H), lambda i: (0, 0)),
        pl.BlockSpec((H, _C), lambda i: (0, 0)),
        pl.BlockSpec((1, _C), lambda i: (0, 0)),
    ],
    out_specs=pl.BlockSpec((_R, _C), lambda i: (i, 0)),
    out_shape=jax.ShapeDtypeStruct((N, _C), jnp.float32),
)


def kernel(x, edge_index, W1, b1, W2, b2, W3, b3):
    ei_p = edge_index.reshape(2, NS, NB_P, EB)
    ei_d = edge_index[1].reshape(NC * NS, NB_D, EB)
    degp = _deg_kernel(ei_d)
    u1 = _stage_pre(x, W1)
    y1, inv = _stage_scale(degp, u1)
    s1 = _prop_kernel(y1, ei_p)
    y2 = _stage_mid(s1, y1, inv, b1.reshape(1, H), W2)
    s2 = _prop_kernel(y2, ei_p)
    return _stage_out(s2, y2, inv, b2.reshape(1, H), W3, b3.reshape(1, _C))


# speedup vs baseline: 1.0582x; 1.0150x over previous
import functools

import jax
import jax.numpy as jnp
from jax import lax
from jax.experimental import pallas as pl
from jax.experimental.pallas import tpu as pltpu
from jax.experimental.pallas import tpu_sc as plsc

N = 10000
E = 320000
H = 128
HW = H // 2
NC = 2
NS = 16
EB = 80
EPT_P = E // NS
NB_P = EPT_P // EB
EPT_D = E // (NC * NS)
NB_D = EPT_D // EB
NP = 10240
RPT = NP // NS
DW = 16

_mesh = plsc.VectorSubcoreMesh(
    core_axis_name="c", subcore_axis_name="s", num_cores=NC, num_subcores=NS
)


@functools.partial(
    pl.kernel,
    out_type=jax.ShapeDtypeStruct((NC, NP, DW), jnp.float32),
    mesh=_mesh,
    compiler_params=pltpu.CompilerParams(use_tc_tiling_on_sc=False),
    scratch_types=[
        pltpu.VMEM((NB_D, EB), jnp.int32),
        pltpu.VMEM((EB, DW), jnp.float32),
        pltpu.VMEM((RPT, DW), jnp.float32),
        pltpu.VMEM_SHARED((NP, DW), jnp.float32),
        pltpu.SemaphoreType.DMA,
    ],
)
def _deg_kernel(dst_hbm, out_hbm, didx, ones_v, zb, acc, sem):
    c = lax.axis_index("c")
    s = lax.axis_index("s")
    w = c * NS + s

    def _fill(i, _):
        ones_v[i, :] = jnp.full((DW,), 1.0, jnp.float32)
        return 0

    lax.fori_loop(0, EB, _fill, 0)

    def _zrow(i, _):
        zb[i, :] = jnp.zeros((DW,), jnp.float32)
        return 0

    lax.fori_loop(0, RPT, _zrow, 0)
    pltpu.sync_copy(zb, acc.at[pl.ds(s * RPT, RPT)])
    pltpu.sync_copy(dst_hbm.at[w], didx)
    plsc.subcore_barrier()

    DK = 25

    def _dgroup(g, _):
        base = g * DK
        for b in range(DK):
            pltpu.make_async_copy(ones_v, acc.at[didx.at[base + b]], sem).start(add=True)
        for b in range(DK):
            pltpu.make_async_copy(ones_v, acc.at[didx.at[0]], sem).wait()
        return 0

    lax.fori_loop(0, NB_D // DK, _dgroup, 0)
    plsc.subcore_barrier()
    pltpu.sync_copy(acc.at[pl.ds(s * RPT, RPT)], out_hbm.at[c, pl.ds(s * RPT, RPT), :])


NBUF = 5
NG = NB_P // NBUF


@functools.partial(
    pl.kernel,
    out_type=jax.ShapeDtypeStruct((NC, NP, HW), jnp.float32),
    mesh=_mesh,
    compiler_params=pltpu.CompilerParams(use_tc_tiling_on_sc=False),
    scratch_types=[
        pltpu.VMEM((NB_P, EB), jnp.int32),
        pltpu.VMEM((NB_P, EB), jnp.int32),
        pltpu.VMEM((NBUF, EB, HW), jnp.float32),
        pltpu.VMEM((RPT // 8, HW), jnp.float32),
        pltpu.VMEM_SHARED((NP, HW), jnp.float32),
        pltpu.SemaphoreType.DMA((NBUF,)),
        pltpu.SemaphoreType.DMA,
    ],
)
def _prop_kernel(y_hbm, ei_hbm, out_hbm, sidx, didx, rows, zb, acc, sem, psem):
    c = lax.axis_index("c")
    s = lax.axis_index("s")

    ps = pltpu.make_async_copy(ei_hbm.at[0, s], sidx, psem)
    ps.start()
    pd = pltpu.make_async_copy(ei_hbm.at[1, s], didx, psem)
    pd.start()

    def _zrow(i, _):
        for j in range(HW // 16):
            zb[i, pl.ds(j * 16, 16)] = jnp.zeros((16,), jnp.float32)
        return 0

    lax.fori_loop(0, RPT // 8, _zrow, 0)
    ps.wait()
    pd.wait()

    ytab = y_hbm.at[c]

    def g_start(b, j):
        pltpu.make_async_copy(ytab.at[sidx.at[j]], rows.at[b], sem.at[b]).start()

    def g_wait(b):
        pltpu.make_async_copy(ytab.at[sidx.at[0]], rows.at[b], sem.at[b]).wait()

    def s_start(b, j):
        pltpu.make_async_copy(rows.at[b], acc.at[didx.at[j]], sem.at[b]).start(add=True)

    def s_wait(b):
        pltpu.make_async_copy(rows.at[b], acc.at[didx.at[0]], sem.at[b]).wait()

    for b in range(NBUF):
        g_start(b, b)
    for r in range(8):
        pltpu.sync_copy(zb, acc.at[pl.ds(s * RPT + r * (RPT // 8), RPT // 8)])
    plsc.subcore_barrier()

    def _group(g, _):
        base = g * NBUF
        for b in range(NBUF):
            g_wait(b)
            s_start(b, base + b)
        for b in range(NBUF):
            s_wait(b)
            g_start(b, base + NBUF + b)
        return 0

    lax.fori_loop(0, NG - 1, _group, 0)
    last = (NG - 1) * NBUF
    for b in range(NBUF):
        g_wait(b)
        s_start(b, last + b)
    for b in range(NBUF):
        s_wait(b)
    plsc.subcore_barrier()
    pltpu.sync_copy(acc.at[pl.ds(s * RPT, RPT)], out_hbm.at[c, pl.ds(s * RPT, RPT), :])


_R = 5000
_C = 40


def _stage_pre_body(x_ref, w_ref, u_ref):
    u_ref[...] = jnp.dot(x_ref[...], w_ref[...], preferred_element_type=jnp.float32)


def _stage_scale_body(dp_ref, u_ref, y_ref, inv_ref):
    deg = dp_ref[0, :, 0:1] + dp_ref[1, :, 0:1] + 1.0
    inv = lax.rsqrt(deg)
    y = u_ref[...] * inv
    y_ref[0] = y[:, :HW]
    y_ref[1] = y[:, HW:]
    inv_ref[...] = inv


def _stage_mid_body(sp_ref, y_ref, inv_ref, b_ref, w_ref, y2_ref):
    s = jnp.concatenate([sp_ref[0], sp_ref[1]], axis=1)
    y = jnp.concatenate([y_ref[0], y_ref[1]], axis=1)
    inv = inv_ref[...]
    h = jnp.maximum(inv * (s + y) + b_ref[...], 0.0)
    y2 = jnp.dot(h, w_ref[...], preferred_element_type=jnp.float32) * inv
    y2_ref[0] = y2[:, :HW]
    y2_ref[1] = y2[:, HW:]


def _stage_out_body(sp_ref, y_ref, inv_ref, b_ref, w3_ref, b3_ref, o_ref):
    s = jnp.concatenate([sp_ref[0], sp_ref[1]], axis=1)
    y = jnp.concatenate([y_ref[0], y_ref[1]], axis=1)
    inv = inv_ref[...]
    h = jnp.maximum(inv * (s + y) + b_ref[...], 0.0)
    logits = jnp.dot(h, w3_ref[...], preferred_element_type=jnp.float32) + b3_ref[...]
    m = jnp.max(logits, axis=1, keepdims=True)
    e = jnp.exp(logits - m)
    o_ref[...] = e / jnp.sum(e, axis=1, keepdims=True)


_stage_pre = pl.pallas_call(
    _stage_pre_body,
    grid=(N // _R,),
    in_specs=[
        pl.BlockSpec((_R, H), lambda i: (i, 0)),
        pl.BlockSpec((H, H), lambda i: (0, 0)),
    ],
    out_specs=pl.BlockSpec((_R, H), lambda i: (i, 0)),
    out_shape=jax.ShapeDtypeStruct((N, H), jnp.float32),
)

_stage_scale = pl.pallas_call(
    _stage_scale_body,
    grid=(N // _R,),
    in_specs=[
        pl.BlockSpec((NC, _R, DW), lambda i: (0, i, 0)),
        pl.BlockSpec((_R, H), lambda i: (i, 0)),
    ],
    out_specs=[
        pl.BlockSpec((NC, _R, HW), lambda i: (0, i, 0)),
        pl.BlockSpec((_R, 1), lambda i: (i, 0)),
    ],
    out_shape=[
        jax.ShapeDtypeStruct((NC, N, HW), jnp.float32),
        jax.ShapeDtypeStruct((N, 1), jnp.float32),
    ],
)

_stage_mid = pl.pallas_call(
    _stage_mid_body,
    grid=(N // _R,),
    in_specs=[
        pl.BlockSpec((NC, _R, HW), lambda i: (0, i, 0)),
        pl.BlockSpec((NC, _R, HW), lambda i: (0, i, 0)),
        pl.BlockSpec((_R, 1), lambda i: (i, 0)),
        pl.BlockSpec((1, H), lambda i: (0, 0)),
        pl.BlockSpec((H, H), lambda i: (0, 0)),
    ],
    out_specs=pl.BlockSpec((NC, _R, HW), lambda i: (0, i, 0)),
    out_shape=jax.ShapeDtypeStruct((NC, N, HW), jnp.float32),
)

_stage_out = pl.pallas_call(
    _stage_out_body,
    grid=(N // _R,),
    in_specs=[
        pl.BlockSpec((NC, _R, HW), lambda i: (0, i, 0)),
        pl.BlockSpec((NC, _R, HW), lambda i: (0, i, 0)),
        pl.BlockSpec((_R, 1), lambda i: (i, 0)),
        pl.BlockSpec((1, H), lambda i: (0, 0)),
        pl.BlockSpec((H, _C), lambda i: (0, 0)),
        pl.BlockSpec((1, _C), lambda i: (0, 0)),
    ],
    out_specs=pl.BlockSpec((_R, _C), lambda i: (i, 0)),
    out_shape=jax.ShapeDtypeStruct((N, _C), jnp.float32),
)


def kernel(x, edge_index, W1, b1, W2, b2, W3, b3):
    ei_p = edge_index.reshape(2, NS, NB_P, EB)
    ei_d = edge_index[1].reshape(NC * NS, NB_D, EB)
    degp = _deg_kernel(ei_d)
    u1 = _stage_pre(x, W1)
    y1, inv = _stage_scale(degp, u1)
    s1 = _prop_kernel(y1, ei_p)
    y2 = _stage_mid(s1, y1, inv, b1.reshape(1, H), W2)
    s2 = _prop_kernel(y2, ei_p)
    return _stage_out(s2, y2, inv, b2.reshape(1, H), W3, b3.reshape(1, _C))
